# baseline (speedup 1.0000x reference)
import functools

import jax
import jax.numpy as jnp
from jax import lax
from jax.experimental import pallas as pl
from jax.experimental.pallas import tpu as pltpu
from jax.experimental.pallas import tpu_sc as plsc

_N = 100000
_NPAD = 100352
_CNODE = _NPAD // 16
_E = 3200000
_B = 4096
_NCHUNK = 25
_EPW = _B * _NCHUNK
_EPAD = 32 * _EPW
_OSL = (2 * _NPAD) // 16

_Q_TH_X = 2.0
_Q_TH_Y = 2.0
_FORCE_RATIO = 0.5
_EPSILON = 0.01

_mesh = plsc.VectorSubcoreMesh(core_axis_name="c", subcore_axis_name="s")


@functools.partial(
    pl.kernel,
    out_type=jax.ShapeDtypeStruct((2, 2 * _NPAD), jnp.float32),
    mesh=_mesh,
    scratch_types=[
        pltpu.VMEM_SHARED((2 * _NPAD,), jnp.float32),
        pltpu.VMEM_SHARED((2 * _NPAD,), jnp.float32),
        pltpu.VMEM((_CNODE,), jnp.float32),
        pltpu.VMEM((_CNODE,), jnp.float32),
        pltpu.VMEM((_B,), jnp.int32),
        pltpu.VMEM((_B,), jnp.int32),
        pltpu.VMEM((_B,), jnp.int32),
        pltpu.VMEM((_B,), jnp.int32),
        pltpu.VMEM((_B,), jnp.float32),
        pltpu.VMEM((_B,), jnp.float32),
        pltpu.VMEM((_B,), jnp.float32),
        pltpu.VMEM((_B,), jnp.float32),
        pltpu.VMEM((_B,), jnp.float32),
        pltpu.VMEM((_B,), jnp.float32),
        pltpu.VMEM((_B,), jnp.float32),
        pltpu.VMEM((_B,), jnp.float32),
    ],
)
def _sc_forces(px, py, sx, sy, ii_h, jj_h, out,
               tab, acc, a_v, b_v,
               ii_v, jj_v, iin_v, jjn_v,
               xi_v, xj_v, yi_v, yj_v,
               fxi_v, fxj_v, fyi_v, fyj_v):
    cid = lax.axis_index("c")
    sid = lax.axis_index("s")
    wid = cid * 16 + sid

    nbase = sid * _CNODE

    pltpu.sync_copy(px.at[pl.ds(nbase, _CNODE)], a_v)
    pltpu.sync_copy(sx.at[pl.ds(nbase, _CNODE)], b_v)

    def _cxy(k, _):
        s = pl.ds(k * 16, 16)
        a_v[s] = a_v[s] + 0.5 * b_v[s]
        return 0

    lax.fori_loop(0, _CNODE // 16, _cxy, 0)
    pltpu.sync_copy(a_v, tab.at[pl.ds(nbase, _CNODE)])

    pltpu.sync_copy(py.at[pl.ds(nbase, _CNODE)], a_v)
    pltpu.sync_copy(sy.at[pl.ds(nbase, _CNODE)], b_v)
    lax.fori_loop(0, _CNODE // 16, _cxy, 0)
    pltpu.sync_copy(a_v, tab.at[pl.ds(_NPAD + nbase, _CNODE)])

    def _zero(k, _):
        a_v[pl.ds(k * 16, 16)] = jnp.zeros((16,), jnp.float32)
        return 0

    lax.fori_loop(0, _CNODE // 16, _zero, 0)
    pltpu.sync_copy(a_v, acc.at[pl.ds(nbase, _CNODE)])
    pltpu.sync_copy(a_v, acc.at[pl.ds(_NPAD + nbase, _CNODE)])

    plsc.subcore_barrier()

    ebase0 = wid * _EPW

    def _chunk(c, _):
        base = ebase0 + c * _B
        pltpu.sync_copy(ii_h.at[pl.ds(base, _B)], ii_v)
        pltpu.sync_copy(jj_h.at[pl.ds(base, _B)], jj_v)

        def _shift(k, _):
            s = pl.ds(k * 16, 16)
            iin_v[s] = ii_v[s] + _NPAD
            jjn_v[s] = jj_v[s] + _NPAD
            return 0

        lax.fori_loop(0, _B // 16, _shift, 0)

        pltpu.sync_copy(tab.at[ii_v], xi_v)
        pltpu.sync_copy(tab.at[jj_v], xj_v)
        pltpu.sync_copy(tab.at[iin_v], yi_v)
        pltpu.sync_copy(tab.at[jjn_v], yj_v)

        def _force(k, _):
            s = pl.ds(k * 16, 16)
            dx = xi_v[s] - xj_v[s]
            dy = yi_v[s] - yj_v[s]
            w = ((jnp.abs(dx) < _Q_TH_X) & (jnp.abs(dy) < _Q_TH_Y)
                 & (ii_v[s] != jj_v[s]))
            f = _FORCE_RATIO / (dx * dx + dy * dy + _EPSILON)
            fx = jnp.where(w, f * dx, 0.0)
            fy = jnp.where(w, f * dy, 0.0)
            fxi_v[s] = fx
            fxj_v[s] = -fx
            fyi_v[s] = fy
            fyj_v[s] = -fy
            return 0

        lax.fori_loop(0, _B // 16, _force, 0)

        pltpu.sync_copy(fxi_v, acc.at[ii_v], add=True)
        pltpu.sync_copy(fxj_v, acc.at[jj_v], add=True)
        pltpu.sync_copy(fyi_v, acc.at[iin_v], add=True)
        pltpu.sync_copy(fyj_v, acc.at[jjn_v], add=True)
        return 0

    lax.fori_loop(0, _NCHUNK, _chunk, 0)

    plsc.subcore_barrier()
    pltpu.sync_copy(acc.at[pl.ds(sid * _OSL, _OSL)],
                    out.at[cid, pl.ds(sid * _OSL, _OSL)])


def _reduce_body(p_ref, o_ref):
    s = p_ref[0, :] + p_ref[1, :]
    o_ref[...] = jnp.reshape(jnp.sum(jnp.abs(s)), (1, 1))


def kernel(pos, node_size_x, node_size_y, potential_collision_map):
    cm = potential_collision_map.astype(jnp.int32)
    px = jnp.pad(pos[:_N], (0, _NPAD - _N))
    py = jnp.pad(pos[_N:2 * _N], (0, _NPAD - _N))
    sx = jnp.pad(node_size_x, (0, _NPAD - _N))
    sy = jnp.pad(node_size_y, (0, _NPAD - _N))
    ii = jnp.pad(cm[:, 0], (0, _EPAD - _E))
    jj = jnp.pad(cm[:, 1], (0, _EPAD - _E))
    partials = _sc_forces(px, py, sx, sy, ii, jj)
    res = pl.pallas_call(
        _reduce_body,
        out_shape=jax.ShapeDtypeStruct((1, 1), jnp.float32),
    )(partials)
    return res[0, 0]


# speedup vs baseline: 2.1735x; 2.1735x over previous
import functools

import jax
import jax.numpy as jnp
from jax import lax
from jax.experimental import pallas as pl
from jax.experimental.pallas import tpu as pltpu
from jax.experimental.pallas import tpu_sc as plsc

_N = 100000
_NPAD = 100352
_CNODE = _NPAD // 16
_E = 3200000
_B = 2048
_NCHUNK = 49
_EPW = _B * _NCHUNK
_EPAD = 32 * _EPW
_OSL = (2 * _NPAD) // 16

_Q_TH_X = 2.0
_Q_TH_Y = 2.0
_FORCE_RATIO = 0.5
_EPSILON = 0.01

_mesh = plsc.VectorSubcoreMesh(core_axis_name="c", subcore_axis_name="s")


@functools.partial(
    pl.kernel,
    out_type=jax.ShapeDtypeStruct((2, 2 * _NPAD), jnp.float32),
    mesh=_mesh,
    scratch_types=[
        pltpu.VMEM_SHARED((2 * _NPAD,), jnp.float32),
        pltpu.VMEM_SHARED((2 * _NPAD,), jnp.float32),
        pltpu.VMEM((_CNODE,), jnp.float32),
        pltpu.VMEM((_CNODE,), jnp.float32),
        pltpu.VMEM((_B,), jnp.int32),
        pltpu.VMEM((_B,), jnp.int32),
        pltpu.VMEM((_B,), jnp.int32),
        pltpu.VMEM((_B,), jnp.int32),
        pltpu.VMEM((_B,), jnp.float32),
        pltpu.VMEM((_B,), jnp.float32),
        pltpu.VMEM((_B,), jnp.float32),
        pltpu.VMEM((_B,), jnp.float32),
        pltpu.VMEM((_B,), jnp.float32),
        pltpu.VMEM((_B,), jnp.float32),
        pltpu.VMEM((_B,), jnp.float32),
        pltpu.VMEM((_B,), jnp.float32),
        pltpu.SemaphoreType.DMA,
        pltpu.SemaphoreType.DMA,
    ],
)
def _sc_forces(px, py, sx, sy, ii_h, jj_h, out,
               tab, acc, a_v, b_v,
               ii_v, jj_v, iin_v, jjn_v,
               xi_v, xj_v, yi_v, yj_v,
               fxi_v, fxj_v, fyi_v, fyj_v, sem_g, sem_s):
    cid = lax.axis_index("c")
    sid = lax.axis_index("s")
    wid = cid * 16 + sid

    nbase = sid * _CNODE

    pltpu.sync_copy(px.at[pl.ds(nbase, _CNODE)], a_v)
    pltpu.sync_copy(sx.at[pl.ds(nbase, _CNODE)], b_v)

    def _cxy(k, _):
        s = pl.ds(k * 16, 16)
        a_v[s] = a_v[s] + 0.5 * b_v[s]
        return 0

    lax.fori_loop(0, _CNODE // 16, _cxy, 0)
    pltpu.sync_copy(a_v, tab.at[pl.ds(nbase, _CNODE)])

    pltpu.sync_copy(py.at[pl.ds(nbase, _CNODE)], a_v)
    pltpu.sync_copy(sy.at[pl.ds(nbase, _CNODE)], b_v)
    lax.fori_loop(0, _CNODE // 16, _cxy, 0)
    pltpu.sync_copy(a_v, tab.at[pl.ds(_NPAD + nbase, _CNODE)])

    def _zero(k, _):
        a_v[pl.ds(k * 16, 16)] = jnp.zeros((16,), jnp.float32)
        return 0

    lax.fori_loop(0, _CNODE // 16, _zero, 0)
    pltpu.sync_copy(a_v, acc.at[pl.ds(nbase, _CNODE)])
    pltpu.sync_copy(a_v, acc.at[pl.ds(_NPAD + nbase, _CNODE)])

    plsc.subcore_barrier()

    ebase0 = wid * _EPW

    def _chunk(c, _):
        base = ebase0 + c * _B
        pltpu.sync_copy(ii_h.at[pl.ds(base, _B)], ii_v)
        pltpu.sync_copy(jj_h.at[pl.ds(base, _B)], jj_v)

        def _shift(k, _):
            s = pl.ds(k * 16, 16)
            iin_v[s] = ii_v[s] + _NPAD
            jjn_v[s] = jj_v[s] + _NPAD
            return 0

        lax.fori_loop(0, _B // 16, _shift, 0)

        g1 = pltpu.async_copy(tab.at[ii_v], xi_v, sem_g)
        g2 = pltpu.async_copy(tab.at[jj_v], xj_v, sem_g)
        g3 = pltpu.async_copy(tab.at[iin_v], yi_v, sem_g)
        g4 = pltpu.async_copy(tab.at[jjn_v], yj_v, sem_g)

        @pl.when(c > 0)
        def _drain_prev_scatters():
            pltpu.make_async_copy(fxi_v, acc.at[ii_v], sem_s).wait()
            pltpu.make_async_copy(fxj_v, acc.at[jj_v], sem_s).wait()
            pltpu.make_async_copy(fyi_v, acc.at[iin_v], sem_s).wait()
            pltpu.make_async_copy(fyj_v, acc.at[jjn_v], sem_s).wait()

        g1.wait()
        g2.wait()
        g3.wait()
        g4.wait()

        def _force(k, _):
            s = pl.ds(k * 16, 16)
            dx = xi_v[s] - xj_v[s]
            dy = yi_v[s] - yj_v[s]
            w = ((jnp.abs(dx) < _Q_TH_X) & (jnp.abs(dy) < _Q_TH_Y)
                 & (ii_v[s] != jj_v[s]))
            f = _FORCE_RATIO / (dx * dx + dy * dy + _EPSILON)
            fx = jnp.where(w, f * dx, 0.0)
            fy = jnp.where(w, f * dy, 0.0)
            fxi_v[s] = fx
            fxj_v[s] = -fx
            fyi_v[s] = fy
            fyj_v[s] = -fy
            return 0

        lax.fori_loop(0, _B // 16, _force, 0)

        pltpu.async_copy(fxi_v, acc.at[ii_v], sem_s, add=True)
        pltpu.async_copy(fxj_v, acc.at[jj_v], sem_s, add=True)
        pltpu.async_copy(fyi_v, acc.at[iin_v], sem_s, add=True)
        pltpu.async_copy(fyj_v, acc.at[jjn_v], sem_s, add=True)
        return 0

    lax.fori_loop(0, _NCHUNK, _chunk, 0)

    pltpu.make_async_copy(fxi_v, acc.at[ii_v], sem_s).wait()
    pltpu.make_async_copy(fxj_v, acc.at[jj_v], sem_s).wait()
    pltpu.make_async_copy(fyi_v, acc.at[iin_v], sem_s).wait()
    pltpu.make_async_copy(fyj_v, acc.at[jjn_v], sem_s).wait()

    plsc.subcore_barrier()
    pltpu.sync_copy(acc.at[pl.ds(sid * _OSL, _OSL)],
                    out.at[cid, pl.ds(sid * _OSL, _OSL)])


def _reduce_body(p_ref, o_ref):
    s = p_ref[0, :] + p_ref[1, :]
    o_ref[...] = jnp.reshape(jnp.sum(jnp.abs(s)), (1, 1))


def kernel(pos, node_size_x, node_size_y, potential_collision_map):
    cm = potential_collision_map.astype(jnp.int32)
    px = jnp.pad(pos[:_N], (0, _NPAD - _N))
    py = jnp.pad(pos[_N:2 * _N], (0, _NPAD - _N))
    sx = jnp.pad(node_size_x, (0, _NPAD - _N))
    sy = jnp.pad(node_size_y, (0, _NPAD - _N))
    ii = jnp.pad(cm[:, 0], (0, _EPAD - _E))
    jj = jnp.pad(cm[:, 1], (0, _EPAD - _E))
    partials = _sc_forces(px, py, sx, sy, ii, jj)
    res = pl.pallas_call(
        _reduce_body,
        out_shape=jax.ShapeDtypeStruct((1, 1), jnp.float32),
    )(partials)
    return res[0, 0]
